# baseline (speedup 1.0000x reference)
import functools

import jax
import jax.numpy as jnp
from jax import lax
from jax.experimental import pallas as pl
from jax.experimental.pallas import tpu as pltpu
from jax.experimental.pallas import tpu_sc as plsc

NODE_NUM = 27
N_PER_TYPE = 8192
IN_DIM = 300
HIDDEN_DIM = 128
OUT_DIM = 64
BATCH = 16

SC_TYPES = 8
TC_TYPES = NODE_NUM - SC_TYPES

NUM_CORES = 2
NUM_SUBCORES = 16
NUM_WORKERS = NUM_CORES * NUM_SUBCORES
RPW = N_PER_TYPE // NUM_WORKERS
LANES = 16
KBLK = 19
ROW_PAD = KBLK * LANES
ACC_W = BATCH * ROW_PAD
GROUP = 16
NGROUP = RPW // GROUP
ROWS_W = RPW * IN_DIM
ROWS_BUF = ROWS_W + 16

CHUNK = 1024
NCHUNK = N_PER_TYPE // CHUNK


def _sc_segment_sums(feat_hbm, seg_hbm, zeros_hbm, out_hbm,
                     rows_v, seg_v, segoff_v, acc_v):
    c = lax.axis_index("c")
    s = lax.axis_index("s")
    slice_id = c * NUM_SUBCORES + s

    iota = lax.iota(jnp.int32, LANES)
    pltpu.sync_copy(zeros_hbm.at[pl.ds(0, 16)], rows_v.at[pl.ds(ROWS_W, 16)])

    def per_type(t, carry):
        base = t * N_PER_TYPE + slice_id * RPW
        pltpu.sync_copy(seg_hbm.at[pl.ds(base, RPW)], seg_v)
        pltpu.sync_copy(zeros_hbm, acc_v)
        pltpu.sync_copy(feat_hbm.at[pl.ds(base * IN_DIM, ROWS_W)],
                        rows_v.at[pl.ds(0, ROWS_W)])

        @plsc.parallel_loop(0, RPW // LANES, unroll=2)
        def prep(i):
            segoff_v[pl.ds(i * LANES, LANES)] = (
                seg_v[pl.ds(i * LANES, LANES)] * ROW_PAD)

        @plsc.parallel_loop(0, NGROUP, unroll=2)
        def per_group(g):
            grow = g * GROUP
            gword = grow * IN_DIM
            for r in range(GROUP):
                soff = plsc.load_gather(
                    segoff_v, [jnp.full((LANES,), grow + r, jnp.int32)])
                rbase = gword + r * IN_DIM
                for k in range(KBLK):
                    v = plsc.load_gather(rows_v, [rbase + k * LANES + iota])
                    plsc.addupdate_scatter(acc_v, [soff + (k * LANES) + iota], v)

        pltpu.sync_copy(acc_v, out_hbm.at[t, slice_id])
        return carry

    lax.fori_loop(0, SC_TYPES, per_type, 0)


_sc_kernel = functools.partial(
    pl.kernel,
    out_type=jax.ShapeDtypeStruct((SC_TYPES, NUM_WORKERS, ACC_W), jnp.float32),
    mesh=plsc.VectorSubcoreMesh(core_axis_name="c", subcore_axis_name="s"),
    compiler_params=pltpu.CompilerParams(needs_layout_passes=False),
    scratch_types=[
        pltpu.VMEM((ROWS_BUF,), jnp.float32),
        pltpu.VMEM((RPW,), jnp.int32),
        pltpu.VMEM((RPW,), jnp.int32),
        pltpu.VMEM((ACC_W,), jnp.float32),
    ],
)(_sc_segment_sums)


def _tc_reduce_mlp_body(part_ref, seg_ref, w1_ref, b1_ref, w2_ref, b2_ref,
                        out_ref):
    sums = jnp.sum(part_ref[0][:, :, :IN_DIM], axis=0)
    seg_row = seg_ref[0, 0, :]
    iota_b = jax.lax.broadcasted_iota(jnp.int32, (BATCH, N_PER_TYPE), 0)
    counts = jnp.sum((seg_row[None, :] == iota_b).astype(jnp.float32),
                     axis=1)
    mean = jnp.where(counts[:, None] > 0,
                     sums / jnp.maximum(counts, 1.0)[:, None],
                     0.0)
    h = jnp.dot(mean.astype(jnp.bfloat16), w1_ref[0].astype(jnp.bfloat16),
                preferred_element_type=jnp.float32) + b1_ref[0]
    h = jnp.maximum(h, 0.0)
    out = jnp.dot(h.astype(jnp.bfloat16), w2_ref[0].astype(jnp.bfloat16),
                  preferred_element_type=jnp.float32) + b2_ref[0]
    out_ref[0] = out


def _tc_matmul_body(seg_ref, feat_ref, w1_ref, b1_ref, w2_ref, b2_ref,
                    out_ref, acc_ref):
    c = pl.program_id(1)
    seg_chunk = seg_ref[0, 0, pl.ds(c * CHUNK, CHUNK)]
    iota = jax.lax.broadcasted_iota(jnp.int32, (BATCH, CHUNK), 0)
    onehot = (seg_chunk[None, :] == iota).astype(jnp.bfloat16)
    partial = jnp.dot(onehot, feat_ref[0].astype(jnp.bfloat16),
                      preferred_element_type=jnp.float32)

    @pl.when(c == 0)
    def _():
        acc_ref[...] = partial

    @pl.when(c > 0)
    def _():
        acc_ref[...] += partial

    @pl.when(c == NCHUNK - 1)
    def _():
        seg_row = seg_ref[0, 0, :]
        iota_b = jax.lax.broadcasted_iota(jnp.int32, (BATCH, N_PER_TYPE), 0)
        counts = jnp.sum((seg_row[None, :] == iota_b).astype(jnp.float32),
                         axis=1)
        mean = jnp.where(counts[:, None] > 0,
                         acc_ref[...] / jnp.maximum(counts, 1.0)[:, None],
                         0.0)
        h = jnp.dot(mean.astype(jnp.bfloat16), w1_ref[0].astype(jnp.bfloat16),
                    preferred_element_type=jnp.float32) + b1_ref[0]
        h = jnp.maximum(h, 0.0)
        out = jnp.dot(h.astype(jnp.bfloat16), w2_ref[0].astype(jnp.bfloat16),
                      preferred_element_type=jnp.float32) + b2_ref[0]
        out_ref[0] = out


@jax.jit
def kernel(feat, segment_ids, W1, b1, W2, b2):
    seg3 = segment_ids.reshape(NODE_NUM, 1, N_PER_TYPE)
    b1r = b1.reshape(NODE_NUM, 1, HIDDEN_DIM)
    b2r = b2.reshape(NODE_NUM, 1, OUT_DIM)

    feat_sc = feat[:SC_TYPES].reshape(SC_TYPES * N_PER_TYPE * IN_DIM)
    seg_sc = segment_ids[:SC_TYPES].reshape(-1).astype(jnp.int32)
    zeros = jnp.zeros((ACC_W,), jnp.float32)
    partials = _sc_kernel(feat_sc, seg_sc, zeros)
    part4 = partials.reshape(SC_TYPES, NUM_WORKERS, BATCH, ROW_PAD)

    out_sc = pl.pallas_call(
        _tc_reduce_mlp_body,
        grid=(SC_TYPES,),
        in_specs=[
            pl.BlockSpec((1, NUM_WORKERS, BATCH, ROW_PAD),
                         lambda t: (t, 0, 0, 0)),
            pl.BlockSpec((1, 1, N_PER_TYPE), lambda t: (t, 0, 0)),
            pl.BlockSpec((1, IN_DIM, HIDDEN_DIM), lambda t: (t, 0, 0)),
            pl.BlockSpec((1, 1, HIDDEN_DIM), lambda t: (t, 0, 0)),
            pl.BlockSpec((1, HIDDEN_DIM, OUT_DIM), lambda t: (t, 0, 0)),
            pl.BlockSpec((1, 1, OUT_DIM), lambda t: (t, 0, 0)),
        ],
        out_specs=pl.BlockSpec((1, BATCH, OUT_DIM), lambda t: (t, 0, 0)),
        out_shape=jax.ShapeDtypeStruct((SC_TYPES, BATCH, OUT_DIM),
                                       jnp.float32),
    )(part4, seg3, W1, b1r, W2, b2r)

    out_tc = pl.pallas_call(
        _tc_matmul_body,
        grid=(TC_TYPES, NCHUNK),
        in_specs=[
            pl.BlockSpec((1, 1, N_PER_TYPE), lambda t, c: (t + SC_TYPES, 0, 0)),
            pl.BlockSpec((1, CHUNK, IN_DIM), lambda t, c: (t + SC_TYPES, c, 0)),
            pl.BlockSpec((1, IN_DIM, HIDDEN_DIM),
                         lambda t, c: (t + SC_TYPES, 0, 0)),
            pl.BlockSpec((1, 1, HIDDEN_DIM), lambda t, c: (t + SC_TYPES, 0, 0)),
            pl.BlockSpec((1, HIDDEN_DIM, OUT_DIM),
                         lambda t, c: (t + SC_TYPES, 0, 0)),
            pl.BlockSpec((1, 1, OUT_DIM), lambda t, c: (t + SC_TYPES, 0, 0)),
        ],
        out_specs=pl.BlockSpec((1, BATCH, OUT_DIM), lambda t, c: (t, 0, 0)),
        out_shape=jax.ShapeDtypeStruct((TC_TYPES, BATCH, OUT_DIM),
                                       jnp.float32),
        scratch_shapes=[pltpu.VMEM((BATCH, IN_DIM), jnp.float32)],
    )(seg3, feat, W1, b1r, W2, b2r)

    out = jnp.concatenate([out_sc, out_tc], axis=0)
    return jnp.transpose(out, (1, 0, 2))


# speedup vs baseline: 1.1555x; 1.1555x over previous
import functools

import jax
import jax.numpy as jnp
from jax import lax
from jax.experimental import pallas as pl
from jax.experimental.pallas import tpu as pltpu
from jax.experimental.pallas import tpu_sc as plsc

NODE_NUM = 27
N_PER_TYPE = 8192
IN_DIM = 300
HIDDEN_DIM = 128
OUT_DIM = 64
BATCH = 16

SC_TYPES = 4
TC_TYPES = NODE_NUM - SC_TYPES

NUM_CORES = 2
NUM_SUBCORES = 16
NUM_WORKERS = NUM_CORES * NUM_SUBCORES
RPW = N_PER_TYPE // NUM_WORKERS
LANES = 16
KBLK = 19
ROW_PAD = KBLK * LANES
ACC_W = BATCH * ROW_PAD
GROUP = 16
NGROUP = RPW // GROUP
ROWS_W = RPW * IN_DIM
ROWS_BUF = ROWS_W + 16

CHUNK = 1024
NCHUNK = N_PER_TYPE // CHUNK


def _sc_segment_sums(feat_hbm, seg_hbm, zeros_hbm, out_hbm,
                     rows_v, seg_v, segoff_v, acc_v):
    c = lax.axis_index("c")
    s = lax.axis_index("s")
    slice_id = c * NUM_SUBCORES + s

    iota = lax.iota(jnp.int32, LANES)
    pltpu.sync_copy(zeros_hbm.at[pl.ds(0, 16)], rows_v.at[pl.ds(ROWS_W, 16)])

    def per_type(t, carry):
        base = t * N_PER_TYPE + slice_id * RPW
        pltpu.sync_copy(seg_hbm.at[pl.ds(base, RPW)], seg_v)
        pltpu.sync_copy(zeros_hbm, acc_v)
        pltpu.sync_copy(feat_hbm.at[pl.ds(base * IN_DIM, ROWS_W)],
                        rows_v.at[pl.ds(0, ROWS_W)])

        @plsc.parallel_loop(0, RPW // LANES, unroll=2)
        def prep(i):
            segoff_v[pl.ds(i * LANES, LANES)] = (
                seg_v[pl.ds(i * LANES, LANES)] * ROW_PAD)

        @plsc.parallel_loop(0, NGROUP, unroll=2)
        def per_group(g):
            grow = g * GROUP
            gword = grow * IN_DIM
            for r in range(GROUP):
                soff = plsc.load_gather(
                    segoff_v, [jnp.full((LANES,), grow + r, jnp.int32)])
                rbase = gword + r * IN_DIM
                for k in range(KBLK):
                    v = plsc.load_gather(rows_v, [rbase + k * LANES + iota])
                    plsc.addupdate_scatter(acc_v, [soff + (k * LANES) + iota], v)

        pltpu.sync_copy(acc_v, out_hbm.at[t, slice_id])
        return carry

    lax.fori_loop(0, SC_TYPES, per_type, 0)


_sc_kernel = functools.partial(
    pl.kernel,
    out_type=jax.ShapeDtypeStruct((SC_TYPES, NUM_WORKERS, ACC_W), jnp.float32),
    mesh=plsc.VectorSubcoreMesh(core_axis_name="c", subcore_axis_name="s"),
    compiler_params=pltpu.CompilerParams(needs_layout_passes=False),
    scratch_types=[
        pltpu.VMEM((ROWS_BUF,), jnp.float32),
        pltpu.VMEM((RPW,), jnp.int32),
        pltpu.VMEM((RPW,), jnp.int32),
        pltpu.VMEM((ACC_W,), jnp.float32),
    ],
)(_sc_segment_sums)


def _tc_reduce_mlp_body(part_ref, seg_ref, w1_ref, b1_ref, w2_ref, b2_ref,
                        out_ref):
    sums = jnp.sum(part_ref[0][:, :, :IN_DIM], axis=0)
    seg_row = seg_ref[0, 0, :]
    iota_b = jax.lax.broadcasted_iota(jnp.int32, (BATCH, N_PER_TYPE), 0)
    counts = jnp.sum((seg_row[None, :] == iota_b).astype(jnp.float32),
                     axis=1)
    mean = jnp.where(counts[:, None] > 0,
                     sums / jnp.maximum(counts, 1.0)[:, None],
                     0.0)
    h = jnp.dot(mean.astype(jnp.bfloat16), w1_ref[0].astype(jnp.bfloat16),
                preferred_element_type=jnp.float32) + b1_ref[0]
    h = jnp.maximum(h, 0.0)
    out = jnp.dot(h.astype(jnp.bfloat16), w2_ref[0].astype(jnp.bfloat16),
                  preferred_element_type=jnp.float32) + b2_ref[0]
    out_ref[0] = out


def _tc_matmul_body(seg_ref, feat_ref, w1_ref, b1_ref, w2_ref, b2_ref,
                    out_ref, acc_ref):
    c = pl.program_id(1)
    seg_chunk = seg_ref[0, 0, pl.ds(c * CHUNK, CHUNK)]
    iota = jax.lax.broadcasted_iota(jnp.int32, (BATCH, CHUNK), 0)
    onehot = (seg_chunk[None, :] == iota).astype(jnp.bfloat16)
    partial = jnp.dot(onehot, feat_ref[0].astype(jnp.bfloat16),
                      preferred_element_type=jnp.float32)

    @pl.when(c == 0)
    def _():
        acc_ref[...] = partial

    @pl.when(c > 0)
    def _():
        acc_ref[...] += partial

    @pl.when(c == NCHUNK - 1)
    def _():
        seg_row = seg_ref[0, 0, :]
        iota_b = jax.lax.broadcasted_iota(jnp.int32, (BATCH, N_PER_TYPE), 0)
        counts = jnp.sum((seg_row[None, :] == iota_b).astype(jnp.float32),
                         axis=1)
        mean = jnp.where(counts[:, None] > 0,
                         acc_ref[...] / jnp.maximum(counts, 1.0)[:, None],
                         0.0)
        h = jnp.dot(mean.astype(jnp.bfloat16), w1_ref[0].astype(jnp.bfloat16),
                    preferred_element_type=jnp.float32) + b1_ref[0]
        h = jnp.maximum(h, 0.0)
        out = jnp.dot(h.astype(jnp.bfloat16), w2_ref[0].astype(jnp.bfloat16),
                      preferred_element_type=jnp.float32) + b2_ref[0]
        out_ref[0] = out


@jax.jit
def kernel(feat, segment_ids, W1, b1, W2, b2):
    seg3 = segment_ids.reshape(NODE_NUM, 1, N_PER_TYPE)
    b1r = b1.reshape(NODE_NUM, 1, HIDDEN_DIM)
    b2r = b2.reshape(NODE_NUM, 1, OUT_DIM)

    feat_sc = feat[:SC_TYPES].reshape(SC_TYPES * N_PER_TYPE * IN_DIM)
    seg_sc = segment_ids[:SC_TYPES].reshape(-1).astype(jnp.int32)
    zeros = jnp.zeros((ACC_W,), jnp.float32)
    partials = _sc_kernel(feat_sc, seg_sc, zeros)
    part4 = partials.reshape(SC_TYPES, NUM_WORKERS, BATCH, ROW_PAD)

    out_sc = pl.pallas_call(
        _tc_reduce_mlp_body,
        grid=(SC_TYPES,),
        in_specs=[
            pl.BlockSpec((1, NUM_WORKERS, BATCH, ROW_PAD),
                         lambda t: (t, 0, 0, 0)),
            pl.BlockSpec((1, 1, N_PER_TYPE), lambda t: (t, 0, 0)),
            pl.BlockSpec((1, IN_DIM, HIDDEN_DIM), lambda t: (t, 0, 0)),
            pl.BlockSpec((1, 1, HIDDEN_DIM), lambda t: (t, 0, 0)),
            pl.BlockSpec((1, HIDDEN_DIM, OUT_DIM), lambda t: (t, 0, 0)),
            pl.BlockSpec((1, 1, OUT_DIM), lambda t: (t, 0, 0)),
        ],
        out_specs=pl.BlockSpec((1, BATCH, OUT_DIM), lambda t: (t, 0, 0)),
        out_shape=jax.ShapeDtypeStruct((SC_TYPES, BATCH, OUT_DIM),
                                       jnp.float32),
    )(part4, seg3, W1, b1r, W2, b2r)

    out_tc = pl.pallas_call(
        _tc_matmul_body,
        grid=(TC_TYPES, NCHUNK),
        in_specs=[
            pl.BlockSpec((1, 1, N_PER_TYPE), lambda t, c: (t + SC_TYPES, 0, 0)),
            pl.BlockSpec((1, CHUNK, IN_DIM), lambda t, c: (t + SC_TYPES, c, 0)),
            pl.BlockSpec((1, IN_DIM, HIDDEN_DIM),
                         lambda t, c: (t + SC_TYPES, 0, 0)),
            pl.BlockSpec((1, 1, HIDDEN_DIM), lambda t, c: (t + SC_TYPES, 0, 0)),
            pl.BlockSpec((1, HIDDEN_DIM, OUT_DIM),
                         lambda t, c: (t + SC_TYPES, 0, 0)),
            pl.BlockSpec((1, 1, OUT_DIM), lambda t, c: (t + SC_TYPES, 0, 0)),
        ],
        out_specs=pl.BlockSpec((1, BATCH, OUT_DIM), lambda t, c: (t, 0, 0)),
        out_shape=jax.ShapeDtypeStruct((TC_TYPES, BATCH, OUT_DIM),
                                       jnp.float32),
        scratch_shapes=[pltpu.VMEM((BATCH, IN_DIM), jnp.float32)],
    )(seg3, feat, W1, b1r, W2, b2r)

    out = jnp.concatenate([out_sc, out_tc], axis=0)
    return jnp.transpose(out, (1, 0, 2))
